# Initial kernel scaffold; baseline (speedup 1.0000x reference)
import jax
import jax.numpy as jnp
from jax.experimental import pallas as pl


def kernel(x, edge_index, Wl1, Wr1, b1, g1, be1, Wl2, Wr2, b2, g2, be2, Wl3, Wr3, b3, g3, be3, W4, b4, g4, be4, W5, b5):
    raise NotImplementedError("write your pallas kernel here")


# speedup vs baseline: 1.5901x; 1.5901x over previous
import functools

import jax
import jax.numpy as jnp
from jax import lax
from jax.experimental import pallas as pl
from jax.experimental.pallas import tpu as pltpu
from jax.experimental.pallas import tpu_sc as plsc

N = 10000
E = 160000
D = 256
H = 256
OUT = 128
EPS = 1e-5
F32 = jnp.float32
I32 = jnp.int32

NC = 2
NS = 16
LANES = 16

CH = 80
EPW = E // NS
NCHUNK = EPW // CH
RPT = 640
RPT_LAST = N - 15 * RPT


def _make_seg_sum(want_deg):
  mesh = plsc.VectorSubcoreMesh(core_axis_name="c", subcore_axis_name="s",
                                num_cores=NC, num_subcores=NS)
  out_type = [
      jax.ShapeDtypeStruct((N, 128), F32),
      jax.ShapeDtypeStruct((N, 128), F32),
  ]
  scratch = [
      pltpu.VMEM((CH,), I32),
      pltpu.VMEM((CH,), I32),
      pltpu.VMEM((CH, 128), F32),
      pltpu.VMEM((16, 128), F32),
      pltpu.VMEM_SHARED((N, 128), F32),
      pltpu.SemaphoreType.DMA,
  ]
  if want_deg:
    out_type.append(jax.ShapeDtypeStruct((80, 128), F32))
    scratch += [
        pltpu.VMEM((80, 128), F32),
        pltpu.VMEM((80,), I32),
        pltpu.VMEM_SHARED((80, 128), F32),
    ]

  def body(*refs):
    if want_deg:
      (xr, srch, dsth, out0, out1, degout,
       src_v, dst_v, rows_v, zbuf, acc, sem, deg_part, idx80, degsh) = refs
    else:
      (xr, srch, dsth, out0, out1,
       src_v, dst_v, rows_v, zbuf, acc, sem) = refs
    c = lax.axis_index("c")
    s = lax.axis_index("s")

    z16f = jnp.zeros((LANES,), F32)
    iota16 = lax.iota(I32, 16)
    for r in range(16):
      for j in range(8):
        zbuf[r, pl.ds(j * 16, 16)] = z16f

    base_r = pl.multiple_of(s * RPT, 8)

    def _zero_rows(dst_ref):
      @pl.when(s < 15)
      def _():
        for i in range(RPT // 16):
          pltpu.sync_copy(zbuf, dst_ref.at[pl.ds(base_r + i * 16, 16)])

      @pl.when(s == 15)
      def _():
        for i in range(RPT_LAST // 16):
          pltpu.sync_copy(zbuf, dst_ref.at[pl.ds(15 * RPT + i * 16, 16)])

    _zero_rows(acc)

    if want_deg:
      one16 = jnp.full((LANES,), 1.0, F32)
      for r in range(80):
        for j in range(8):
          deg_part[r, pl.ds(j * 16, 16)] = z16f
      for j in range(5):
        idx80[pl.ds(j * 16, 16)] = iota16 + j * 16

      @pl.when((c == 0) & (s == 0))
      def _():
        for i in range(5):
          pltpu.sync_copy(zbuf, degsh.at[pl.ds(i * 16, 16)])

    plsc.subcore_barrier()

    ebase = s * EPW

    def _chunk(k, _):
      off = pl.multiple_of(ebase + k * CH, 8)
      pltpu.sync_copy(srch.at[pl.ds(off, CH)], src_v)
      pltpu.sync_copy(dsth.at[pl.ds(off, CH)], dst_v)
      if want_deg:
        @pl.when(c == 0)
        def _():
          for j in range(CH // 16):
            d = dst_v[pl.ds(j * 16, 16)]
            plsc.addupdate_scatter(deg_part, [d // 128, d % 128], one16)
      for j in range(CH // 16):
        sl = pl.ds(j * 16, 16)
        src_v[sl] = src_v[sl] * 2 + c
      pltpu.async_copy(xr.at[src_v], rows_v, sem).wait()
      pltpu.sync_copy(rows_v, acc.at[dst_v], add=True)
      return 0

    lax.fori_loop(0, NCHUNK, _chunk, 0)

    plsc.subcore_barrier()

    def _copy_out(src_ref, dst_ref):
      @pl.when(s < 15)
      def _():
        pltpu.sync_copy(src_ref.at[pl.ds(base_r, RPT)],
                        dst_ref.at[pl.ds(base_r, RPT)])

      @pl.when(s == 15)
      def _():
        pltpu.sync_copy(src_ref.at[pl.ds(15 * RPT, RPT_LAST)],
                        dst_ref.at[pl.ds(15 * RPT, RPT_LAST)])

    @pl.when(c == 0)
    def _():
      _copy_out(acc, out0)

    @pl.when(c == 1)
    def _():
      _copy_out(acc, out1)

    if want_deg:
      @pl.when(c == 0)
      def _():
        pltpu.sync_copy(deg_part, degsh.at[idx80], add=True)

      plsc.subcore_barrier()

      @pl.when((c == 0) & (s == 0))
      def _():
        pltpu.sync_copy(degsh, degout)

  return pl.kernel(body, out_type=tuple(out_type), mesh=mesh,
                   scratch_types=tuple(scratch),
                   compiler_params=pltpu.CompilerParams(
                       needs_layout_passes=False))


NW = NC * NS
RW = 320
NPAD = NW * RW
SCH = 2000
NSCH = E // SCH
GM = 32


def _make_seg_max():
  mesh = plsc.VectorSubcoreMesh(core_axis_name="c", subcore_axis_name="s", num_cores=NC, num_subcores=NS)
  out_type = jax.ShapeDtypeStruct((NPAD, 256), F32)
  scratch = [
      pltpu.VMEM((SCH,), I32),
      pltpu.VMEM((SCH,), I32),
      pltpu.VMEM((SCH + 16,), I32),
      pltpu.VMEM((SCH + 16,), I32),
      pltpu.VMEM((GM, 256), F32),
      pltpu.VMEM((RW + 8, 256), F32),
      pltpu.SemaphoreType.DMA,
  ]

  def body(h1, srch, dsth, out,
           src_v, dst_v, csrc, cldst, rows_v, acc, sem):
    c = lax.axis_index("c")
    s = lax.axis_index("s")
    wid = s * NC + c
    base = pl.multiple_of(wid * RW, 8)

    z16f = jnp.zeros((LANES,), F32)
    iota16 = lax.iota(I32, 16)

    def _zero_row(r, _):
      rowi = jnp.full((LANES,), r, I32)
      for j in range(16):
        plsc.store_scatter(acc, [rowi, iota16 + j * 16], z16f)
      return 0

    lax.fori_loop(0, RW + 8, _zero_row, 0)
    for j in range((SCH + 16) // 16):
      csrc[pl.ds(j * 16, 16)] = iota16 + (j * 16)

    def scan_chunk(k, _):
      off = k * SCH
      pltpu.sync_copy(srch.at[pl.ds(off, SCH)], src_v)
      pltpu.sync_copy(dsth.at[pl.ds(off, SCH)], dst_v)
      cnt = jnp.int32(0)
      for j in range(SCH // 16):
        sl = pl.ds(j * 16, 16)
        d = dst_v[sl]
        sv = src_v[sl]
        m = (d >= base) & (d < base + RW)
        cum = plsc.cumsum(m.astype(I32))
        pos = cnt + cum - 1
        plsc.store_scatter(csrc, [pos], sv, mask=m)
        plsc.store_scatter(cldst, [pos], d - base, mask=m)
        cnt = cnt + jnp.max(cum)

      def rmw_chunk(g, _):
        gb = g * GM
        pltpu.async_copy(h1.at[csrc.at[pl.ds(gb, GM)]], rows_v, sem).wait()
        cntv = jnp.full((LANES,), cnt, I32)
        trash = jnp.full((LANES,), RW, I32)
        for l in range(GM):
          eidx = jnp.full((LANES,), gb + l, I32)
          ldv = plsc.load_gather(cldst, [eidx])
          rowv = jnp.where(eidx < cntv, ldv, trash)
          msgv = jnp.full((LANES,), l, I32)
          for f in range(16):
            cols = iota16 + (f * 16)
            a = plsc.load_gather(acc, [rowv, cols])
            mg = plsc.load_gather(rows_v, [msgv, cols])
            plsc.store_scatter(acc, [rowv, cols], jnp.maximum(a, mg))
        return 0

      ng = (cnt + (GM - 1)) // GM
      lax.fori_loop(0, ng, rmw_chunk, 0)
      return 0

    lax.fori_loop(0, NSCH, scan_chunk, 0)
    pltpu.sync_copy(acc.at[pl.ds(0, RW)], out.at[pl.ds(base, RW)])

  return pl.kernel(body, out_type=out_type, mesh=mesh,
                   scratch_types=tuple(scratch),
                   compiler_params=pltpu.CompilerParams(
                       needs_layout_passes=False))


BR = 1000
GRID = N // BR
_DN = (((1,), (1,)), ((), ()))


def _dot(a, w):
  return lax.dot_general(a, w, _DN, preferred_element_type=F32)


def _stats_write(st_ref, sacc):
  ri = lax.broadcasted_iota(I32, (8, 128), 0)
  ci = lax.broadcasted_iota(I32, (8, 128), 1)
  st_ref[...] = jnp.where((ri == 0) & (ci == 0), sacc[0],
                          jnp.where((ri == 0) & (ci == 1), sacc[1], 0.0))


def _accum_stats(i, z, st_ref, sacc):
  bs = jnp.sum(z)
  bq = jnp.sum(z * z)

  @pl.when(i == 0)
  def _():
    sacc[0] = bs
    sacc[1] = bq

  @pl.when(i > 0)
  def _():
    sacc[0] = sacc[0] + bs
    sacc[1] = sacc[1] + bq

  @pl.when(i == GRID - 1)
  def _():
    _stats_write(st_ref, sacc)


def _m1_body(a0, a1, xin, dg, wl, wr, bb, z_ref, st_ref, sacc):
  i = pl.program_id(0)
  inv = 1.0 / jnp.maximum(dg[...], 1.0)
  wlv = wl[...]
  z = (_dot(a0[...] * inv, wlv[:, :128]) + _dot(a1[...] * inv, wlv[:, 128:])
       + _dot(xin[...], wr[...]) + bb[...])
  z_ref[...] = z
  _accum_stats(i, z, st_ref, sacc)


def _m2_body(ag, h1, wl, wr, bb, z_ref, st_ref, sacc):
  i = pl.program_id(0)
  z = _dot(ag[...], wl[...]) + _dot(h1[...], wr[...]) + bb[...]
  z_ref[...] = z
  _accum_stats(i, z, st_ref, sacc)


def _m3_body(a0, a1, h2, dg, wr, bb, z_ref, st_ref, sacc):
  i = pl.program_id(0)
  inv = 1.0 / jnp.maximum(dg[...], 1.0)
  mean = jnp.concatenate([a0[...], a1[...]], axis=1) * inv
  z = mean + _dot(h2[...], wr[...]) + bb[...]
  z_ref[...] = z
  _accum_stats(i, z, st_ref, sacc)


def _a1_body(z, g, be, scal, h_ref):
  mu = scal[0]
  r = scal[1]
  h_ref[...] = jnp.maximum((z[...] - mu) * r * g[...] + be[...], 0.0)


def _a2p3_body(z, g, be, wl3, scal, h_ref, p_ref):
  mu = scal[0]
  r = scal[1]
  hh = jnp.maximum((z[...] - mu) * r * g[...] + be[...], 0.0)
  h_ref[...] = hh
  p_ref[...] = _dot(hh, wl3[...])


def _f_body(z3, g3, be3, scal, w4, b4, g4, be4, w5, b5, out_ref):
  mu = scal[0]
  r = scal[1]
  h3 = jnp.maximum((z3[...] - mu) * r * g3[...] + be3[...], 0.0)
  t = _dot(h3, w4[...]) + b4[...]
  mu_r = jnp.mean(t, axis=1, keepdims=True)
  xc = t - mu_r
  var = jnp.mean(xc * xc, axis=1, keepdims=True)
  h4 = jnp.maximum(xc / jnp.sqrt(var + EPS) * g4[...] + be4[...], 0.0)
  out_ref[...] = _dot(h4, w5[...]) + b5[...]


def _rows(bd):
  return pl.BlockSpec((BR, bd), lambda i: (i, 0))


def _full(shape):
  return pl.BlockSpec(shape, lambda i: tuple(0 for _ in shape))


_SMEM_SPEC = pl.BlockSpec(memory_space=pltpu.SMEM)
_ST_SHAPE = jax.ShapeDtypeStruct((8, 128), F32)
_ST_SPEC = pl.BlockSpec((8, 128), lambda i: (0, 0))
_SACC = [pltpu.SMEM((2,), F32)]


def _finish_stats(st, cnt):
  mu = st[0, 0] / cnt
  var = jnp.maximum(st[0, 1] / cnt - mu * mu, 0.0)
  r = 1.0 / (jnp.sqrt(var) + EPS)
  return jnp.stack([mu, r])


@jax.jit
def kernel(x, edge_index, Wl1, Wr1, b1, g1, be1, Wl2, Wr2, b2, g2, be2,
           Wl3, Wr3, b3, g3, be3, W4, b4, g4, be4, W5, b5):
  src = edge_index[0]
  dst = edge_index[1]
  xr = x.reshape(2 * N, 128)

  a10, a11, deg80 = _make_seg_sum(True)(xr, src, dst)
  deg = deg80.reshape(80 * 128)[:N].reshape(N, 1)

  z1, st1 = pl.pallas_call(
      _m1_body,
      grid=(GRID,),
      in_specs=[_rows(128), _rows(128), _rows(256), _rows(1),
                _full((H, D)), _full((H, D)), _full((1, H))],
      out_specs=[_rows(H), _ST_SPEC],
      out_shape=[jax.ShapeDtypeStruct((N, H), F32), _ST_SHAPE],
      scratch_shapes=_SACC,
  )(a10, a11, x, deg, Wl1, Wr1, b1.reshape(1, H))

  h1 = pl.pallas_call(
      _a1_body,
      grid=(GRID,),
      in_specs=[_rows(H), _full((1, H)), _full((1, H)), _SMEM_SPEC],
      out_specs=_rows(H),
      out_shape=jax.ShapeDtypeStruct((N, H), F32),
  )(z1, g1.reshape(1, H), be1.reshape(1, H), _finish_stats(st1, N * H))

  amax = _make_seg_max()(h1, src, dst)

  z2, st2 = pl.pallas_call(
      _m2_body,
      grid=(GRID,),
      in_specs=[_rows(256), _rows(H), _full((2 * H, H)), _full((2 * H, H)),
                _full((1, 2 * H))],
      out_specs=[_rows(2 * H), _ST_SPEC],
      out_shape=[jax.ShapeDtypeStruct((N, 2 * H), F32), _ST_SHAPE],
      scratch_shapes=_SACC,
  )(amax, h1, Wl2, Wr2, b2.reshape(1, 2 * H))

  h2, p2 = pl.pallas_call(
      _a2p3_body,
      grid=(GRID,),
      in_specs=[_rows(2 * H), _full((1, 2 * H)), _full((1, 2 * H)),
                _full((H, 2 * H)), _SMEM_SPEC],
      out_specs=[_rows(2 * H), _rows(H)],
      out_shape=[jax.ShapeDtypeStruct((N, 2 * H), F32),
                 jax.ShapeDtypeStruct((N, H), F32)],
  )(z2, g2.reshape(1, 2 * H), be2.reshape(1, 2 * H), Wl3,
    _finish_stats(st2, N * 2 * H))

  ap0, ap1 = _make_seg_sum(False)(p2.reshape(2 * N, 128), src, dst)

  z3, st3 = pl.pallas_call(
      _m3_body,
      grid=(GRID,),
      in_specs=[_rows(128), _rows(128), _rows(2 * H), _rows(1),
                _full((H, 2 * H)), _full((1, H))],
      out_specs=[_rows(H), _ST_SPEC],
      out_shape=[jax.ShapeDtypeStruct((N, H), F32), _ST_SHAPE],
      scratch_shapes=_SACC,
  )(ap0, ap1, h2, deg, Wr3, b3.reshape(1, H))

  out = pl.pallas_call(
      _f_body,
      grid=(GRID,),
      in_specs=[_rows(H), _full((1, H)), _full((1, H)), _SMEM_SPEC,
                _full((H // 2, H)), _full((1, H // 2)), _full((1, H // 2)),
                _full((1, H // 2)), _full((OUT, H // 2)), _full((1, OUT))],
      out_specs=_rows(OUT),
      out_shape=jax.ShapeDtypeStruct((N, OUT), F32),
  )(z3, g3.reshape(1, H), be3.reshape(1, H), _finish_stats(st3, N * H),
    W4, b4.reshape(1, H // 2), g4.reshape(1, H // 2), be4.reshape(1, H // 2),
    W5, b5.reshape(1, OUT))

  return out


# speedup vs baseline: 1.5997x; 1.0060x over previous
import functools

import jax
import jax.numpy as jnp
from jax import lax
from jax.experimental import pallas as pl
from jax.experimental.pallas import tpu as pltpu
from jax.experimental.pallas import tpu_sc as plsc

N = 10000
E = 160000
D = 256
H = 256
OUT = 128
EPS = 1e-5
F32 = jnp.float32
I32 = jnp.int32

NC = 2
NS = 16
LANES = 16

CH = 80
EPW = E // NS
NCHUNK = EPW // CH
RPT = 640
RPT_LAST = N - 15 * RPT


def _make_seg_sum(want_deg):
  mesh = plsc.VectorSubcoreMesh(core_axis_name="c", subcore_axis_name="s",
                                num_cores=NC, num_subcores=NS)
  out_type = [
      jax.ShapeDtypeStruct((N, 128), F32),
      jax.ShapeDtypeStruct((N, 128), F32),
  ]
  scratch = [
      pltpu.VMEM((CH,), I32),
      pltpu.VMEM((CH,), I32),
      pltpu.VMEM((CH, 128), F32),
      pltpu.VMEM((16, 128), F32),
      pltpu.VMEM_SHARED((N, 128), F32),
      pltpu.SemaphoreType.DMA,
  ]
  if want_deg:
    out_type.append(jax.ShapeDtypeStruct((80, 128), F32))
    scratch += [
        pltpu.VMEM((80, 128), F32),
        pltpu.VMEM((80,), I32),
        pltpu.VMEM_SHARED((80, 128), F32),
    ]

  def body(*refs):
    if want_deg:
      (xr, srch, dsth, out0, out1, degout,
       src_v, dst_v, rows_v, zbuf, acc, sem, deg_part, idx80, degsh) = refs
    else:
      (xr, srch, dsth, out0, out1,
       src_v, dst_v, rows_v, zbuf, acc, sem) = refs
    c = lax.axis_index("c")
    s = lax.axis_index("s")

    z16f = jnp.zeros((LANES,), F32)
    iota16 = lax.iota(I32, 16)
    for r in range(16):
      for j in range(8):
        zbuf[r, pl.ds(j * 16, 16)] = z16f

    base_r = pl.multiple_of(s * RPT, 8)

    def _zero_rows(dst_ref):
      @pl.when(s < 15)
      def _():
        for i in range(RPT // 16):
          pltpu.sync_copy(zbuf, dst_ref.at[pl.ds(base_r + i * 16, 16)])

      @pl.when(s == 15)
      def _():
        for i in range(RPT_LAST // 16):
          pltpu.sync_copy(zbuf, dst_ref.at[pl.ds(15 * RPT + i * 16, 16)])

    _zero_rows(acc)

    if want_deg:
      one16 = jnp.full((LANES,), 1.0, F32)
      for r in range(80):
        for j in range(8):
          deg_part[r, pl.ds(j * 16, 16)] = z16f
      for j in range(5):
        idx80[pl.ds(j * 16, 16)] = iota16 + j * 16

      @pl.when((c == 0) & (s == 0))
      def _():
        for i in range(5):
          pltpu.sync_copy(zbuf, degsh.at[pl.ds(i * 16, 16)])

    plsc.subcore_barrier()

    ebase = s * EPW

    def _chunk(k, _):
      off = pl.multiple_of(ebase + k * CH, 8)
      pltpu.sync_copy(srch.at[pl.ds(off, CH)], src_v)
      pltpu.sync_copy(dsth.at[pl.ds(off, CH)], dst_v)
      if want_deg:
        @pl.when(c == 0)
        def _():
          for j in range(CH // 16):
            d = dst_v[pl.ds(j * 16, 16)]
            plsc.addupdate_scatter(deg_part, [d // 128, d % 128], one16)
      for j in range(CH // 16):
        sl = pl.ds(j * 16, 16)
        src_v[sl] = src_v[sl] * 2 + c
      pltpu.async_copy(xr.at[src_v], rows_v, sem).wait()
      pltpu.sync_copy(rows_v, acc.at[dst_v], add=True)
      return 0

    lax.fori_loop(0, NCHUNK, _chunk, 0)

    plsc.subcore_barrier()

    def _copy_out(src_ref, dst_ref):
      @pl.when(s < 15)
      def _():
        pltpu.sync_copy(src_ref.at[pl.ds(base_r, RPT)],
                        dst_ref.at[pl.ds(base_r, RPT)])

      @pl.when(s == 15)
      def _():
        pltpu.sync_copy(src_ref.at[pl.ds(15 * RPT, RPT_LAST)],
                        dst_ref.at[pl.ds(15 * RPT, RPT_LAST)])

    @pl.when(c == 0)
    def _():
      _copy_out(acc, out0)

    @pl.when(c == 1)
    def _():
      _copy_out(acc, out1)

    if want_deg:
      @pl.when(c == 0)
      def _():
        pltpu.sync_copy(deg_part, degsh.at[idx80], add=True)

      plsc.subcore_barrier()

      @pl.when((c == 0) & (s == 0))
      def _():
        pltpu.sync_copy(degsh, degout)

  return pl.kernel(body, out_type=tuple(out_type), mesh=mesh,
                   scratch_types=tuple(scratch),
                   compiler_params=pltpu.CompilerParams(
                       needs_layout_passes=False))


NW = NC * NS
RW = 320
NPAD = NW * RW
SCH = 2000
NSCH = E // SCH
GM = 64


def _make_seg_max():
  mesh = plsc.VectorSubcoreMesh(core_axis_name="c", subcore_axis_name="s", num_cores=NC, num_subcores=NS)
  out_type = jax.ShapeDtypeStruct((NPAD, 256), F32)
  scratch = [
      pltpu.VMEM((SCH,), I32),
      pltpu.VMEM((SCH,), I32),
      pltpu.VMEM((SCH + 16,), I32),
      pltpu.VMEM((SCH + 16,), I32),
      pltpu.VMEM((GM, 256), F32),
      pltpu.VMEM((RW + 8, 256), F32),
      pltpu.SemaphoreType.DMA,
  ]

  def body(h1, srch, dsth, out,
           src_v, dst_v, csrc, cldst, rows_v, acc, sem):
    c = lax.axis_index("c")
    s = lax.axis_index("s")
    wid = s * NC + c
    base = pl.multiple_of(wid * RW, 8)

    z16f = jnp.zeros((LANES,), F32)
    iota16 = lax.iota(I32, 16)

    def _zero_row(r, _):
      rowi = jnp.full((LANES,), r, I32)
      for j in range(16):
        plsc.store_scatter(acc, [rowi, iota16 + j * 16], z16f)
      return 0

    lax.fori_loop(0, RW + 8, _zero_row, 0)
    for j in range((SCH + 16) // 16):
      csrc[pl.ds(j * 16, 16)] = iota16 + (j * 16)

    def scan_chunk(k, _):
      off = k * SCH
      pltpu.sync_copy(srch.at[pl.ds(off, SCH)], src_v)
      pltpu.sync_copy(dsth.at[pl.ds(off, SCH)], dst_v)
      cntv = jnp.zeros((LANES,), I32)
      for j in range(SCH // 16):
        sl = pl.ds(j * 16, 16)
        d = dst_v[sl]
        sv = src_v[sl]
        m = (d >= base) & (d < base + RW)
        cum = plsc.cumsum(m.astype(I32))
        pos = cntv + cum - 1
        plsc.store_scatter(csrc, [pos], sv, mask=m)
        plsc.store_scatter(cldst, [pos], d - base, mask=m)
        cntv = cntv + plsc.all_reduce_population_count(m)
      cnt = jnp.max(cntv)

      def rmw_chunk(g, _):
        gb = g * GM
        pltpu.async_copy(h1.at[csrc.at[pl.ds(gb, GM)]], rows_v, sem).wait()
        cntsp = jnp.full((LANES,), cnt, I32)
        trash = jnp.full((LANES,), RW, I32)
        for l in range(GM):
          eidx = jnp.full((LANES,), gb + l, I32)
          ldv = plsc.load_gather(cldst, [eidx])
          rowv = jnp.where(eidx < cntsp, ldv, trash)
          for f in range(16):
            cols = iota16 + (f * 16)
            a = plsc.load_gather(acc, [rowv, cols])
            mg = rows_v[l, pl.ds(f * 16, 16)]
            plsc.store_scatter(acc, [rowv, cols], jnp.maximum(a, mg))
        return 0

      ng = (cnt + (GM - 1)) // GM
      lax.fori_loop(0, ng, rmw_chunk, 0)
      return 0

    lax.fori_loop(0, NSCH, scan_chunk, 0)
    pltpu.sync_copy(acc.at[pl.ds(0, RW)], out.at[pl.ds(base, RW)])

  return pl.kernel(body, out_type=out_type, mesh=mesh,
                   scratch_types=tuple(scratch),
                   compiler_params=pltpu.CompilerParams(
                       needs_layout_passes=False))


BR = 1000
GRID = N // BR
_DN = (((1,), (1,)), ((), ()))


def _dot(a, w):
  return lax.dot_general(a, w, _DN, preferred_element_type=F32)


def _stats_write(st_ref, sacc):
  ri = lax.broadcasted_iota(I32, (8, 128), 0)
  ci = lax.broadcasted_iota(I32, (8, 128), 1)
  st_ref[...] = jnp.where((ri == 0) & (ci == 0), sacc[0],
                          jnp.where((ri == 0) & (ci == 1), sacc[1], 0.0))


def _accum_stats(i, z, st_ref, sacc):
  bs = jnp.sum(z)
  bq = jnp.sum(z * z)

  @pl.when(i == 0)
  def _():
    sacc[0] = bs
    sacc[1] = bq

  @pl.when(i > 0)
  def _():
    sacc[0] = sacc[0] + bs
    sacc[1] = sacc[1] + bq

  @pl.when(i == GRID - 1)
  def _():
    _stats_write(st_ref, sacc)


def _m1_body(a0, a1, xin, dg, wl, wr, bb, z_ref, st_ref, sacc):
  i = pl.program_id(0)
  inv = 1.0 / jnp.maximum(dg[...], 1.0)
  wlv = wl[...]
  z = (_dot(a0[...] * inv, wlv[:, :128]) + _dot(a1[...] * inv, wlv[:, 128:])
       + _dot(xin[...], wr[...]) + bb[...])
  z_ref[...] = z
  _accum_stats(i, z, st_ref, sacc)


def _m2_body(ag, h1, wl, wr, bb, z_ref, st_ref, sacc):
  i = pl.program_id(0)
  z = _dot(ag[...], wl[...]) + _dot(h1[...], wr[...]) + bb[...]
  z_ref[...] = z
  _accum_stats(i, z, st_ref, sacc)


def _m3_body(a0, a1, h2, dg, wr, bb, z_ref, st_ref, sacc):
  i = pl.program_id(0)
  inv = 1.0 / jnp.maximum(dg[...], 1.0)
  mean = jnp.concatenate([a0[...], a1[...]], axis=1) * inv
  z = mean + _dot(h2[...], wr[...]) + bb[...]
  z_ref[...] = z
  _accum_stats(i, z, st_ref, sacc)


def _a1_body(z, g, be, scal, h_ref):
  mu = scal[0]
  r = scal[1]
  h_ref[...] = jnp.maximum((z[...] - mu) * r * g[...] + be[...], 0.0)


def _a2p3_body(z, g, be, wl3, scal, h_ref, p_ref):
  mu = scal[0]
  r = scal[1]
  hh = jnp.maximum((z[...] - mu) * r * g[...] + be[...], 0.0)
  h_ref[...] = hh
  p_ref[...] = _dot(hh, wl3[...])


def _f_body(z3, g3, be3, scal, w4, b4, g4, be4, w5, b5, out_ref):
  mu = scal[0]
  r = scal[1]
  h3 = jnp.maximum((z3[...] - mu) * r * g3[...] + be3[...], 0.0)
  t = _dot(h3, w4[...]) + b4[...]
  mu_r = jnp.mean(t, axis=1, keepdims=True)
  xc = t - mu_r
  var = jnp.mean(xc * xc, axis=1, keepdims=True)
  h4 = jnp.maximum(xc / jnp.sqrt(var + EPS) * g4[...] + be4[...], 0.0)
  out_ref[...] = _dot(h4, w5[...]) + b5[...]


def _rows(bd):
  return pl.BlockSpec((BR, bd), lambda i: (i, 0))


def _full(shape):
  return pl.BlockSpec(shape, lambda i: tuple(0 for _ in shape))


_SMEM_SPEC = pl.BlockSpec(memory_space=pltpu.SMEM)
_ST_SHAPE = jax.ShapeDtypeStruct((8, 128), F32)
_ST_SPEC = pl.BlockSpec((8, 128), lambda i: (0, 0))
_SACC = [pltpu.SMEM((2,), F32)]


def _finish_stats(st, cnt):
  mu = st[0, 0] / cnt
  var = jnp.maximum(st[0, 1] / cnt - mu * mu, 0.0)
  r = 1.0 / (jnp.sqrt(var) + EPS)
  return jnp.stack([mu, r])


@jax.jit
def kernel(x, edge_index, Wl1, Wr1, b1, g1, be1, Wl2, Wr2, b2, g2, be2,
           Wl3, Wr3, b3, g3, be3, W4, b4, g4, be4, W5, b5):
  src = edge_index[0]
  dst = edge_index[1]
  xr = x.reshape(2 * N, 128)

  a10, a11, deg80 = _make_seg_sum(True)(xr, src, dst)
  deg = deg80.reshape(80 * 128)[:N].reshape(N, 1)

  z1, st1 = pl.pallas_call(
      _m1_body,
      grid=(GRID,),
      in_specs=[_rows(128), _rows(128), _rows(256), _rows(1),
                _full((H, D)), _full((H, D)), _full((1, H))],
      out_specs=[_rows(H), _ST_SPEC],
      out_shape=[jax.ShapeDtypeStruct((N, H), F32), _ST_SHAPE],
      scratch_shapes=_SACC,
  )(a10, a11, x, deg, Wl1, Wr1, b1.reshape(1, H))

  h1 = pl.pallas_call(
      _a1_body,
      grid=(GRID,),
      in_specs=[_rows(H), _full((1, H)), _full((1, H)), _SMEM_SPEC],
      out_specs=_rows(H),
      out_shape=jax.ShapeDtypeStruct((N, H), F32),
  )(z1, g1.reshape(1, H), be1.reshape(1, H), _finish_stats(st1, N * H))

  amax = _make_seg_max()(h1, src, dst)

  z2, st2 = pl.pallas_call(
      _m2_body,
      grid=(GRID,),
      in_specs=[_rows(256), _rows(H), _full((2 * H, H)), _full((2 * H, H)),
                _full((1, 2 * H))],
      out_specs=[_rows(2 * H), _ST_SPEC],
      out_shape=[jax.ShapeDtypeStruct((N, 2 * H), F32), _ST_SHAPE],
      scratch_shapes=_SACC,
  )(amax, h1, Wl2, Wr2, b2.reshape(1, 2 * H))

  h2, p2 = pl.pallas_call(
      _a2p3_body,
      grid=(GRID,),
      in_specs=[_rows(2 * H), _full((1, 2 * H)), _full((1, 2 * H)),
                _full((H, 2 * H)), _SMEM_SPEC],
      out_specs=[_rows(2 * H), _rows(H)],
      out_shape=[jax.ShapeDtypeStruct((N, 2 * H), F32),
                 jax.ShapeDtypeStruct((N, H), F32)],
  )(z2, g2.reshape(1, 2 * H), be2.reshape(1, 2 * H), Wl3,
    _finish_stats(st2, N * 2 * H))

  ap0, ap1 = _make_seg_sum(False)(p2.reshape(2 * N, 128), src, dst)

  z3, st3 = pl.pallas_call(
      _m3_body,
      grid=(GRID,),
      in_specs=[_rows(128), _rows(128), _rows(2 * H), _rows(1),
                _full((H, 2 * H)), _full((1, H))],
      out_specs=[_rows(H), _ST_SPEC],
      out_shape=[jax.ShapeDtypeStruct((N, H), F32), _ST_SHAPE],
      scratch_shapes=_SACC,
  )(ap0, ap1, h2, deg, Wr3, b3.reshape(1, H))

  out = pl.pallas_call(
      _f_body,
      grid=(GRID,),
      in_specs=[_rows(H), _full((1, H)), _full((1, H)), _SMEM_SPEC,
                _full((H // 2, H)), _full((1, H // 2)), _full((1, H // 2)),
                _full((1, H // 2)), _full((OUT, H // 2)), _full((1, OUT))],
      out_specs=_rows(OUT),
      out_shape=jax.ShapeDtypeStruct((N, OUT), F32),
  )(z3, g3.reshape(1, H), be3.reshape(1, H), _finish_stats(st3, N * H),
    W4, b4.reshape(1, H // 2), g4.reshape(1, H // 2), be4.reshape(1, H // 2),
    W5, b5.reshape(1, OUT))

  return out


# speedup vs baseline: 2.0287x; 1.2681x over previous
import functools

import jax
import jax.numpy as jnp
from jax import lax
from jax.experimental import pallas as pl
from jax.experimental.pallas import tpu as pltpu
from jax.experimental.pallas import tpu_sc as plsc

N = 10000
E = 160000
D = 256
H = 256
OUT = 128
EPS = 1e-5
F32 = jnp.float32
I32 = jnp.int32

NC = 2
NS = 16
LANES = 16

CH = 80
EPW = E // NS
NCHUNK = EPW // CH
RPT = 640
RPT_LAST = N - 15 * RPT


def _make_seg_sum(want_deg):
  mesh = plsc.VectorSubcoreMesh(core_axis_name="c", subcore_axis_name="s",
                                num_cores=NC, num_subcores=NS)
  out_type = [
      jax.ShapeDtypeStruct((N, 128), F32),
      jax.ShapeDtypeStruct((N, 128), F32),
  ]
  scratch = [
      pltpu.VMEM((CH,), I32),
      pltpu.VMEM((CH,), I32),
      pltpu.VMEM((CH, 128), F32),
      pltpu.VMEM((16, 128), F32),
      pltpu.VMEM_SHARED((N, 128), F32),
      pltpu.SemaphoreType.DMA,
  ]
  if want_deg:
    out_type.append(jax.ShapeDtypeStruct((80, 128), F32))
    scratch += [
        pltpu.VMEM((80, 128), F32),
        pltpu.VMEM((80,), I32),
        pltpu.VMEM_SHARED((80, 128), F32),
    ]

  def body(*refs):
    if want_deg:
      (xr, srch, dsth, out0, out1, degout,
       src_v, dst_v, rows_v, zbuf, acc, sem, deg_part, idx80, degsh) = refs
    else:
      (xr, srch, dsth, out0, out1,
       src_v, dst_v, rows_v, zbuf, acc, sem) = refs
    c = lax.axis_index("c")
    s = lax.axis_index("s")

    z16f = jnp.zeros((LANES,), F32)
    iota16 = lax.iota(I32, 16)
    for r in range(16):
      for j in range(8):
        zbuf[r, pl.ds(j * 16, 16)] = z16f

    base_r = pl.multiple_of(s * RPT, 8)

    def _zero_rows(dst_ref):
      @pl.when(s < 15)
      def _():
        for i in range(RPT // 16):
          pltpu.sync_copy(zbuf, dst_ref.at[pl.ds(base_r + i * 16, 16)])

      @pl.when(s == 15)
      def _():
        for i in range(RPT_LAST // 16):
          pltpu.sync_copy(zbuf, dst_ref.at[pl.ds(15 * RPT + i * 16, 16)])

    _zero_rows(acc)

    if want_deg:
      one16 = jnp.full((LANES,), 1.0, F32)
      for r in range(80):
        for j in range(8):
          deg_part[r, pl.ds(j * 16, 16)] = z16f
      for j in range(5):
        idx80[pl.ds(j * 16, 16)] = iota16 + j * 16

      @pl.when((c == 0) & (s == 0))
      def _():
        for i in range(5):
          pltpu.sync_copy(zbuf, degsh.at[pl.ds(i * 16, 16)])

    plsc.subcore_barrier()

    ebase = s * EPW

    def _chunk(k, _):
      off = pl.multiple_of(ebase + k * CH, 8)
      pltpu.sync_copy(srch.at[pl.ds(off, CH)], src_v)
      pltpu.sync_copy(dsth.at[pl.ds(off, CH)], dst_v)
      if want_deg:
        @pl.when(c == 0)
        def _():
          for j in range(CH // 16):
            d = dst_v[pl.ds(j * 16, 16)]
            plsc.addupdate_scatter(deg_part, [d // 128, d % 128], one16)
      for j in range(CH // 16):
        sl = pl.ds(j * 16, 16)
        src_v[sl] = src_v[sl] * 2 + c
      pltpu.async_copy(xr.at[src_v], rows_v, sem).wait()
      pltpu.sync_copy(rows_v, acc.at[dst_v], add=True)
      return 0

    lax.fori_loop(0, NCHUNK, _chunk, 0)

    plsc.subcore_barrier()

    def _copy_out(src_ref, dst_ref):
      @pl.when(s < 15)
      def _():
        pltpu.sync_copy(src_ref.at[pl.ds(base_r, RPT)],
                        dst_ref.at[pl.ds(base_r, RPT)])

      @pl.when(s == 15)
      def _():
        pltpu.sync_copy(src_ref.at[pl.ds(15 * RPT, RPT_LAST)],
                        dst_ref.at[pl.ds(15 * RPT, RPT_LAST)])

    @pl.when(c == 0)
    def _():
      _copy_out(acc, out0)

    @pl.when(c == 1)
    def _():
      _copy_out(acc, out1)

    if want_deg:
      @pl.when(c == 0)
      def _():
        pltpu.sync_copy(deg_part, degsh.at[idx80], add=True)

      plsc.subcore_barrier()

      @pl.when((c == 0) & (s == 0))
      def _():
        pltpu.sync_copy(degsh, degout)

  return pl.kernel(body, out_type=tuple(out_type), mesh=mesh,
                   scratch_types=tuple(scratch),
                   compiler_params=pltpu.CompilerParams(
                       needs_layout_passes=False))


NW = NC * NS
RW = 320
NPAD = NW * RW
SCH = 2000
NSCH = E // SCH
GM = 32


def _make_seg_max():
  mesh = plsc.VectorSubcoreMesh(core_axis_name="c", subcore_axis_name="s", num_cores=NC, num_subcores=NS)
  out_type = jax.ShapeDtypeStruct((NPAD, 256), F32)
  scratch = [
      pltpu.VMEM((SCH,), I32),
      pltpu.VMEM((SCH,), I32),
      pltpu.VMEM((SCH + 16,), I32),
      pltpu.VMEM((SCH + 16,), I32),
      pltpu.VMEM((GM, 256), F32),
      pltpu.VMEM((RW + 8, 256), F32),
      pltpu.SemaphoreType.DMA,
  ]

  def body(h1, srch, dsth, out,
           src_v, dst_v, csrc, cldst, rows_v, acc, sem):
    c = lax.axis_index("c")
    s = lax.axis_index("s")
    wid = s * NC + c
    base = pl.multiple_of(wid * RW, 8)

    z16f = jnp.zeros((LANES,), F32)
    iota16 = lax.iota(I32, 16)

    def _zero_row(r, _):
      rowi = jnp.full((LANES,), r, I32)
      for j in range(16):
        plsc.store_scatter(acc, [rowi, iota16 + j * 16], z16f)
      return 0

    lax.fori_loop(0, RW + 8, _zero_row, 0)
    def _initc(j, _):
      plsc.store_scatter(csrc, [iota16 + j * 16], iota16 + j * 16)
      return 0

    lax.fori_loop(0, (SCH + 16) // 16, _initc, 0)

    def scan_chunk(k, _):
      off = k * SCH
      pltpu.sync_copy(srch.at[pl.ds(off, SCH)], src_v)
      pltpu.sync_copy(dsth.at[pl.ds(off, SCH)], dst_v)
      def sgroup(j, cntv):
        idxv = iota16 + j * 16
        d = plsc.load_gather(dst_v, [idxv])
        sv = plsc.load_gather(src_v, [idxv])
        m = (d >= base) & (d < base + RW)
        cum = plsc.cumsum(m.astype(I32))
        pos = cntv + cum - 1
        plsc.store_scatter(csrc, [pos], sv, mask=m)
        plsc.store_scatter(cldst, [pos], d - base, mask=m)
        return cntv + plsc.all_reduce_population_count(m)

      cntv = lax.fori_loop(0, SCH // 16, sgroup, jnp.zeros((LANES,), I32))
      cnt = jnp.max(cntv)

      def rmw_chunk(g, _):
        gb = g * GM
        pltpu.async_copy(h1.at[csrc.at[pl.ds(gb, GM)]], rows_v, sem).wait()
        cntsp = jnp.full((LANES,), cnt, I32)
        trash = jnp.full((LANES,), RW, I32)
        for l in range(GM):
          eidx = jnp.full((LANES,), gb + l, I32)
          ldv = plsc.load_gather(cldst, [eidx])
          rowv = jnp.where(eidx < cntsp, ldv, trash)
          for f in range(16):
            cols = iota16 + (f * 16)
            a = plsc.load_gather(acc, [rowv, cols])
            mg = rows_v[l, pl.ds(f * 16, 16)]
            plsc.store_scatter(acc, [rowv, cols], jnp.maximum(a, mg))
        return 0

      ng = (cnt + (GM - 1)) // GM
      lax.fori_loop(0, ng, rmw_chunk, 0)
      return 0

    lax.fori_loop(0, NSCH, scan_chunk, 0)
    pltpu.sync_copy(acc.at[pl.ds(0, RW)], out.at[pl.ds(base, RW)])

  return pl.kernel(body, out_type=out_type, mesh=mesh,
                   scratch_types=tuple(scratch),
                   compiler_params=pltpu.CompilerParams(
                       needs_layout_passes=False))


BR = 1000
GRID = N // BR
_DN = (((1,), (1,)), ((), ()))


def _dot(a, w):
  return lax.dot_general(a, w, _DN, preferred_element_type=F32)


def _stats_write(st_ref, sacc):
  ri = lax.broadcasted_iota(I32, (8, 128), 0)
  ci = lax.broadcasted_iota(I32, (8, 128), 1)
  st_ref[...] = jnp.where((ri == 0) & (ci == 0), sacc[0],
                          jnp.where((ri == 0) & (ci == 1), sacc[1], 0.0))


def _accum_stats(i, z, st_ref, sacc):
  bs = jnp.sum(z)
  bq = jnp.sum(z * z)

  @pl.when(i == 0)
  def _():
    sacc[0] = bs
    sacc[1] = bq

  @pl.when(i > 0)
  def _():
    sacc[0] = sacc[0] + bs
    sacc[1] = sacc[1] + bq

  @pl.when(i == GRID - 1)
  def _():
    _stats_write(st_ref, sacc)


def _m1_body(a0, a1, xin, dg, wl, wr, bb, z_ref, st_ref, sacc):
  i = pl.program_id(0)
  inv = 1.0 / jnp.maximum(dg[...], 1.0)
  wlv = wl[...]
  z = (_dot(a0[...] * inv, wlv[:, :128]) + _dot(a1[...] * inv, wlv[:, 128:])
       + _dot(xin[...], wr[...]) + bb[...])
  z_ref[...] = z
  _accum_stats(i, z, st_ref, sacc)


def _m2_body(ag, h1, wl, wr, bb, z_ref, st_ref, sacc):
  i = pl.program_id(0)
  z = _dot(ag[...], wl[...]) + _dot(h1[...], wr[...]) + bb[...]
  z_ref[...] = z
  _accum_stats(i, z, st_ref, sacc)


def _m3_body(a0, a1, h2, dg, wr, bb, z_ref, st_ref, sacc):
  i = pl.program_id(0)
  inv = 1.0 / jnp.maximum(dg[...], 1.0)
  mean = jnp.concatenate([a0[...], a1[...]], axis=1) * inv
  z = mean + _dot(h2[...], wr[...]) + bb[...]
  z_ref[...] = z
  _accum_stats(i, z, st_ref, sacc)


def _a1_body(z, g, be, scal, h_ref):
  mu = scal[0]
  r = scal[1]
  h_ref[...] = jnp.maximum((z[...] - mu) * r * g[...] + be[...], 0.0)


def _a2p3_body(z, g, be, wl3, scal, h_ref, p_ref):
  mu = scal[0]
  r = scal[1]
  hh = jnp.maximum((z[...] - mu) * r * g[...] + be[...], 0.0)
  h_ref[...] = hh
  p_ref[...] = _dot(hh, wl3[...])


def _f_body(z3, g3, be3, scal, w4, b4, g4, be4, w5, b5, out_ref):
  mu = scal[0]
  r = scal[1]
  h3 = jnp.maximum((z3[...] - mu) * r * g3[...] + be3[...], 0.0)
  t = _dot(h3, w4[...]) + b4[...]
  mu_r = jnp.mean(t, axis=1, keepdims=True)
  xc = t - mu_r
  var = jnp.mean(xc * xc, axis=1, keepdims=True)
  h4 = jnp.maximum(xc / jnp.sqrt(var + EPS) * g4[...] + be4[...], 0.0)
  out_ref[...] = _dot(h4, w5[...]) + b5[...]


def _rows(bd):
  return pl.BlockSpec((BR, bd), lambda i: (i, 0))


def _full(shape):
  return pl.BlockSpec(shape, lambda i: tuple(0 for _ in shape))


_SMEM_SPEC = pl.BlockSpec(memory_space=pltpu.SMEM)
_ST_SHAPE = jax.ShapeDtypeStruct((8, 128), F32)
_ST_SPEC = pl.BlockSpec((8, 128), lambda i: (0, 0))
_SACC = [pltpu.SMEM((2,), F32)]


def _finish_stats(st, cnt):
  mu = st[0, 0] / cnt
  var = jnp.maximum(st[0, 1] / cnt - mu * mu, 0.0)
  r = 1.0 / (jnp.sqrt(var) + EPS)
  return jnp.stack([mu, r])


@jax.jit
def kernel(x, edge_index, Wl1, Wr1, b1, g1, be1, Wl2, Wr2, b2, g2, be2,
           Wl3, Wr3, b3, g3, be3, W4, b4, g4, be4, W5, b5):
  src = edge_index[0]
  dst = edge_index[1]
  xr = x.reshape(2 * N, 128)

  a10, a11, deg80 = _make_seg_sum(True)(xr, src, dst)
  deg = deg80.reshape(80 * 128)[:N].reshape(N, 1)

  z1, st1 = pl.pallas_call(
      _m1_body,
      grid=(GRID,),
      in_specs=[_rows(128), _rows(128), _rows(256), _rows(1),
                _full((H, D)), _full((H, D)), _full((1, H))],
      out_specs=[_rows(H), _ST_SPEC],
      out_shape=[jax.ShapeDtypeStruct((N, H), F32), _ST_SHAPE],
      scratch_shapes=_SACC,
  )(a10, a11, x, deg, Wl1, Wr1, b1.reshape(1, H))

  h1 = pl.pallas_call(
      _a1_body,
      grid=(GRID,),
      in_specs=[_rows(H), _full((1, H)), _full((1, H)), _SMEM_SPEC],
      out_specs=_rows(H),
      out_shape=jax.ShapeDtypeStruct((N, H), F32),
  )(z1, g1.reshape(1, H), be1.reshape(1, H), _finish_stats(st1, N * H))

  amax = _make_seg_max()(h1, src, dst)

  z2, st2 = pl.pallas_call(
      _m2_body,
      grid=(GRID,),
      in_specs=[_rows(256), _rows(H), _full((2 * H, H)), _full((2 * H, H)),
                _full((1, 2 * H))],
      out_specs=[_rows(2 * H), _ST_SPEC],
      out_shape=[jax.ShapeDtypeStruct((N, 2 * H), F32), _ST_SHAPE],
      scratch_shapes=_SACC,
  )(amax, h1, Wl2, Wr2, b2.reshape(1, 2 * H))

  h2, p2 = pl.pallas_call(
      _a2p3_body,
      grid=(GRID,),
      in_specs=[_rows(2 * H), _full((1, 2 * H)), _full((1, 2 * H)),
                _full((H, 2 * H)), _SMEM_SPEC],
      out_specs=[_rows(2 * H), _rows(H)],
      out_shape=[jax.ShapeDtypeStruct((N, 2 * H), F32),
                 jax.ShapeDtypeStruct((N, H), F32)],
  )(z2, g2.reshape(1, 2 * H), be2.reshape(1, 2 * H), Wl3,
    _finish_stats(st2, N * 2 * H))

  ap0, ap1 = _make_seg_sum(False)(p2.reshape(2 * N, 128), src, dst)

  z3, st3 = pl.pallas_call(
      _m3_body,
      grid=(GRID,),
      in_specs=[_rows(128), _rows(128), _rows(2 * H), _rows(1),
                _full((H, 2 * H)), _full((1, H))],
      out_specs=[_rows(H), _ST_SPEC],
      out_shape=[jax.ShapeDtypeStruct((N, H), F32), _ST_SHAPE],
      scratch_shapes=_SACC,
  )(ap0, ap1, h2, deg, Wr3, b3.reshape(1, H))

  out = pl.pallas_call(
      _f_body,
      grid=(GRID,),
      in_specs=[_rows(H), _full((1, H)), _full((1, H)), _SMEM_SPEC,
                _full((H // 2, H)), _full((1, H // 2)), _full((1, H // 2)),
                _full((1, H // 2)), _full((OUT, H // 2)), _full((1, OUT))],
      out_specs=_rows(OUT),
      out_shape=jax.ShapeDtypeStruct((N, OUT), F32),
  )(z3, g3.reshape(1, H), be3.reshape(1, H), _finish_stats(st3, N * H),
    W4, b4.reshape(1, H // 2), g4.reshape(1, H // 2), be4.reshape(1, H // 2),
    W5, b5.reshape(1, OUT))

  return out


# speedup vs baseline: 2.0459x; 1.0085x over previous
import jax
import jax.numpy as jnp
from jax import lax
from jax.experimental import pallas as pl
from jax.experimental.pallas import tpu as pltpu
from jax.experimental.pallas import tpu_sc as plsc

N = 10000
E = 160000
D = 256
H = 256
OUT = 128
EPS = 1e-5
F32 = jnp.float32
I32 = jnp.int32

NC = 2
NS = 16
LANES = 16

CH = 80
EPW = E // NS
NCHUNK = EPW // CH
RPT = 640
RPT_LAST = N - 15 * RPT


def _make_seg_sum(want_deg):
  mesh = plsc.VectorSubcoreMesh(core_axis_name="c", subcore_axis_name="s",
                                num_cores=NC, num_subcores=NS)
  out_type = [
      jax.ShapeDtypeStruct((N, 128), F32),
      jax.ShapeDtypeStruct((N, 128), F32),
  ]
  scratch = [
      pltpu.VMEM((CH,), I32),
      pltpu.VMEM((CH,), I32),
      pltpu.VMEM((CH, 128), F32),
      pltpu.VMEM((16, 128), F32),
      pltpu.VMEM_SHARED((N, 128), F32),
      pltpu.SemaphoreType.DMA,
  ]
  if want_deg:
    out_type.append(jax.ShapeDtypeStruct((80, 128), F32))
    scratch += [
        pltpu.VMEM((80, 128), F32),
        pltpu.VMEM((80,), I32),
        pltpu.VMEM_SHARED((80, 128), F32),
    ]

  def body(*refs):
    if want_deg:
      (xr, srch, dsth, out0, out1, degout,
       src_v, dst_v, rows_v, zbuf, acc, sem, deg_part, idx80, degsh) = refs
    else:
      (xr, srch, dsth, out0, out1,
       src_v, dst_v, rows_v, zbuf, acc, sem) = refs
    c = lax.axis_index("c")
    s = lax.axis_index("s")

    z16f = jnp.zeros((LANES,), F32)
    iota16 = lax.iota(I32, 16)
    for r in range(16):
      for j in range(8):
        zbuf[r, pl.ds(j * 16, 16)] = z16f

    base_r = pl.multiple_of(s * RPT, 8)

    def _zero_rows(dst_ref):
      @pl.when(s < 15)
      def _():
        for i in range(RPT // 16):
          pltpu.sync_copy(zbuf, dst_ref.at[pl.ds(base_r + i * 16, 16)])

      @pl.when(s == 15)
      def _():
        for i in range(RPT_LAST // 16):
          pltpu.sync_copy(zbuf, dst_ref.at[pl.ds(15 * RPT + i * 16, 16)])

    _zero_rows(acc)

    if want_deg:
      one16 = jnp.full((LANES,), 1.0, F32)
      for r in range(80):
        for j in range(8):
          deg_part[r, pl.ds(j * 16, 16)] = z16f
      for j in range(5):
        idx80[pl.ds(j * 16, 16)] = iota16 + j * 16

      @pl.when((c == 0) & (s == 0))
      def _():
        for i in range(5):
          pltpu.sync_copy(zbuf, degsh.at[pl.ds(i * 16, 16)])

    plsc.subcore_barrier()

    ebase = s * EPW

    def _chunk(k, _):
      off = pl.multiple_of(ebase + k * CH, 8)
      pltpu.sync_copy(srch.at[pl.ds(off, CH)], src_v)
      pltpu.sync_copy(dsth.at[pl.ds(off, CH)], dst_v)
      if want_deg:
        @pl.when(c == 0)
        def _():
          for j in range(CH // 16):
            d = dst_v[pl.ds(j * 16, 16)]
            plsc.addupdate_scatter(deg_part, [d // 128, d % 128], one16)
      for j in range(CH // 16):
        sl = pl.ds(j * 16, 16)
        src_v[sl] = src_v[sl] * 2 + c
      pltpu.async_copy(xr.at[src_v], rows_v, sem).wait()
      pltpu.sync_copy(rows_v, acc.at[dst_v], add=True)
      return 0

    lax.fori_loop(0, NCHUNK, _chunk, 0)

    plsc.subcore_barrier()

    def _copy_out(src_ref, dst_ref):
      @pl.when(s < 15)
      def _():
        pltpu.sync_copy(src_ref.at[pl.ds(base_r, RPT)],
                        dst_ref.at[pl.ds(base_r, RPT)])

      @pl.when(s == 15)
      def _():
        pltpu.sync_copy(src_ref.at[pl.ds(15 * RPT, RPT_LAST)],
                        dst_ref.at[pl.ds(15 * RPT, RPT_LAST)])

    @pl.when(c == 0)
    def _():
      _copy_out(acc, out0)

    @pl.when(c == 1)
    def _():
      _copy_out(acc, out1)

    if want_deg:
      @pl.when(c == 0)
      def _():
        pltpu.sync_copy(deg_part, degsh.at[idx80], add=True)

      plsc.subcore_barrier()

      @pl.when((c == 0) & (s == 0))
      def _():
        pltpu.sync_copy(degsh, degout)

  return pl.kernel(body, out_type=tuple(out_type), mesh=mesh,
                   scratch_types=tuple(scratch),
                   compiler_params=pltpu.CompilerParams(
                       needs_layout_passes=False))


NW = NC * NS
RW = 320
NPAD = NW * RW
SCH = 2000
NSCH = E // SCH
GM = 32


def _make_seg_max():
  mesh = plsc.VectorSubcoreMesh(core_axis_name="c", subcore_axis_name="s", num_cores=NC, num_subcores=NS)
  out_type = jax.ShapeDtypeStruct((NPAD, 256), F32)
  scratch = [
      pltpu.VMEM((SCH,), I32),
      pltpu.VMEM((SCH,), I32),
      pltpu.VMEM((SCH + 16,), I32),
      pltpu.VMEM((SCH + 16,), I32),
      pltpu.VMEM((GM, 256), F32),
      pltpu.VMEM((RW + 8, 256), F32),
      pltpu.SemaphoreType.DMA,
  ]

  def body(h1, srch, dsth, out,
           src_v, dst_v, csrc, cldst, rows_v, acc, sem):
    c = lax.axis_index("c")
    s = lax.axis_index("s")
    wid = s * NC + c
    base = pl.multiple_of(wid * RW, 8)

    z16f = jnp.zeros((LANES,), F32)
    iota16 = lax.iota(I32, 16)

    def _zero_row(r, _):
      rowi = jnp.full((LANES,), r, I32)
      for j in range(16):
        plsc.store_scatter(acc, [rowi, iota16 + j * 16], z16f)
      return 0

    lax.fori_loop(0, RW + 8, _zero_row, 0)
    def _initc(j, _):
      plsc.store_scatter(csrc, [iota16 + j * 16], iota16 + j * 16)
      return 0

    lax.fori_loop(0, (SCH + 16) // 16, _initc, 0)

    def scan_chunk(k, _):
      off = k * SCH
      pltpu.sync_copy(srch.at[pl.ds(off, SCH)], src_v)
      pltpu.sync_copy(dsth.at[pl.ds(off, SCH)], dst_v)
      def sgroup(j, cntv):
        idxv = iota16 + j * 16
        d = plsc.load_gather(dst_v, [idxv])
        sv = plsc.load_gather(src_v, [idxv])
        m = (d >= base) & (d < base + RW)
        cum = plsc.cumsum(m.astype(I32))
        pos = cntv + cum - 1
        plsc.store_scatter(csrc, [pos], sv, mask=m)
        plsc.store_scatter(cldst, [pos], d - base, mask=m)
        return cntv + plsc.all_reduce_population_count(m)

      cntv = lax.fori_loop(0, SCH // 16, sgroup, jnp.zeros((LANES,), I32))
      cnt = jnp.max(cntv)

      def rmw_chunk(g, _):
        gb = g * GM
        pltpu.async_copy(h1.at[csrc.at[pl.ds(gb, GM)]], rows_v, sem).wait()
        cntsp = jnp.full((LANES,), cnt, I32)
        trash = jnp.full((LANES,), RW, I32)
        for l in range(GM):
          eidx = jnp.full((LANES,), gb + l, I32)
          ldv = plsc.load_gather(cldst, [eidx])
          rowv = jnp.where(eidx < cntsp, ldv, trash)
          for f in range(16):
            cols = iota16 + (f * 16)
            a = plsc.load_gather(acc, [rowv, cols])
            mg = rows_v[l, pl.ds(f * 16, 16)]
            plsc.store_scatter(acc, [rowv, cols], jnp.maximum(a, mg))
        return 0

      ng = (cnt + (GM - 1)) // GM
      lax.fori_loop(0, ng, rmw_chunk, 0)
      return 0

    lax.fori_loop(0, NSCH, scan_chunk, 0)
    pltpu.sync_copy(acc.at[pl.ds(0, RW)], out.at[pl.ds(base, RW)])

  return pl.kernel(body, out_type=out_type, mesh=mesh,
                   scratch_types=tuple(scratch),
                   compiler_params=pltpu.CompilerParams(
                       needs_layout_passes=False))


BR = 1000
GRID = N // BR
_DN = (((1,), (1,)), ((), ()))


def _dot(a, w):
  return lax.dot_general(a, w, _DN, preferred_element_type=F32)


def _stats_write(st_ref, sacc):
  ri = lax.broadcasted_iota(I32, (8, 128), 0)
  ci = lax.broadcasted_iota(I32, (8, 128), 1)
  st_ref[...] = jnp.where((ri == 0) & (ci == 0), sacc[0],
                          jnp.where((ri == 0) & (ci == 1), sacc[1], 0.0))


def _accum_stats(i, z, st_ref, sacc):
  bs = jnp.sum(z)
  bq = jnp.sum(z * z)

  @pl.when(i == 0)
  def _():
    sacc[0] = bs
    sacc[1] = bq

  @pl.when(i > 0)
  def _():
    sacc[0] = sacc[0] + bs
    sacc[1] = sacc[1] + bq

  @pl.when(i == GRID - 1)
  def _():
    _stats_write(st_ref, sacc)


def _m1_body(a0, a1, xin, dg, wl, wr, bb, z_ref, st_ref, sacc):
  i = pl.program_id(0)
  inv = 1.0 / jnp.maximum(dg[...], 1.0)
  wlv = wl[...]
  z = (_dot(a0[...] * inv, wlv[:, :128]) + _dot(a1[...] * inv, wlv[:, 128:])
       + _dot(xin[...], wr[...]) + bb[...])
  z_ref[...] = z
  _accum_stats(i, z, st_ref, sacc)


def _m2_body(ag, h1, wl, wr, bb, z_ref, st_ref, sacc):
  i = pl.program_id(0)
  z = _dot(ag[...], wl[...]) + _dot(h1[...], wr[...]) + bb[...]
  z_ref[...] = z
  _accum_stats(i, z, st_ref, sacc)


def _m3_body(a0, a1, h2, dg, wr, bb, z_ref, st_ref, sacc):
  i = pl.program_id(0)
  inv = 1.0 / jnp.maximum(dg[...], 1.0)
  mean = jnp.concatenate([a0[...], a1[...]], axis=1) * inv
  z = mean + _dot(h2[...], wr[...]) + bb[...]
  z_ref[...] = z
  _accum_stats(i, z, st_ref, sacc)


def _a1_body(z, g, be, scal, h_ref):
  mu = scal[0]
  r = scal[1]
  h_ref[...] = jnp.maximum((z[...] - mu) * r * g[...] + be[...], 0.0)


def _a2p3_body(z, g, be, wl3, scal, h_ref, p_ref):
  mu = scal[0]
  r = scal[1]
  hh = jnp.maximum((z[...] - mu) * r * g[...] + be[...], 0.0)
  h_ref[...] = hh
  p_ref[...] = _dot(hh, wl3[...])


def _f_body(z3, g3, be3, scal, w4, b4, g4, be4, w5, b5, out_ref):
  mu = scal[0]
  r = scal[1]
  h3 = jnp.maximum((z3[...] - mu) * r * g3[...] + be3[...], 0.0)
  t = _dot(h3, w4[...]) + b4[...]
  mu_r = jnp.mean(t, axis=1, keepdims=True)
  xc = t - mu_r
  var = jnp.mean(xc * xc, axis=1, keepdims=True)
  h4 = jnp.maximum(xc / jnp.sqrt(var + EPS) * g4[...] + be4[...], 0.0)
  out_ref[...] = _dot(h4, w5[...]) + b5[...]


def _rows(bd):
  return pl.BlockSpec((BR, bd), lambda i: (i, 0))


def _full(shape):
  return pl.BlockSpec(shape, lambda i: tuple(0 for _ in shape))


_SMEM_SPEC = pl.BlockSpec(memory_space=pltpu.SMEM)
_ST_SHAPE = jax.ShapeDtypeStruct((8, 128), F32)
_ST_SPEC = pl.BlockSpec((8, 128), lambda i: (0, 0))
_SACC = [pltpu.SMEM((2,), F32)]


def _finish_stats(st, cnt):
  mu = st[0, 0] / cnt
  var = jnp.maximum(st[0, 1] / cnt - mu * mu, 0.0)
  r = 1.0 / (jnp.sqrt(var) + EPS)
  return jnp.stack([mu, r])


@jax.jit
def kernel(x, edge_index, Wl1, Wr1, b1, g1, be1, Wl2, Wr2, b2, g2, be2,
           Wl3, Wr3, b3, g3, be3, W4, b4, g4, be4, W5, b5):
  src = edge_index[0]
  dst = edge_index[1]
  xr = x.reshape(2 * N, 128)

  a10, a11, deg80 = _make_seg_sum(True)(xr, src, dst)
  deg = deg80.reshape(80 * 128)[:N].reshape(N, 1)

  z1, st1 = pl.pallas_call(
      _m1_body,
      grid=(GRID,),
      in_specs=[_rows(128), _rows(128), _rows(256), _rows(1),
                _full((H, D)), _full((H, D)), _full((1, H))],
      out_specs=[_rows(H), _ST_SPEC],
      out_shape=[jax.ShapeDtypeStruct((N, H), F32), _ST_SHAPE],
      scratch_shapes=_SACC,
  )(a10, a11, x, deg, Wl1, Wr1, b1.reshape(1, H))

  h1 = pl.pallas_call(
      _a1_body,
      grid=(GRID,),
      in_specs=[_rows(H), _full((1, H)), _full((1, H)), _SMEM_SPEC],
      out_specs=_rows(H),
      out_shape=jax.ShapeDtypeStruct((N, H), F32),
  )(z1, g1.reshape(1, H), be1.reshape(1, H), _finish_stats(st1, N * H))

  amax = _make_seg_max()(h1, src, dst)

  z2, st2 = pl.pallas_call(
      _m2_body,
      grid=(GRID,),
      in_specs=[_rows(256), _rows(H), _full((2 * H, H)), _full((2 * H, H)),
                _full((1, 2 * H))],
      out_specs=[_rows(2 * H), _ST_SPEC],
      out_shape=[jax.ShapeDtypeStruct((N, 2 * H), F32), _ST_SHAPE],
      scratch_shapes=_SACC,
  )(amax, h1, Wl2, Wr2, b2.reshape(1, 2 * H))

  h2, p2 = pl.pallas_call(
      _a2p3_body,
      grid=(GRID,),
      in_specs=[_rows(2 * H), _full((1, 2 * H)), _full((1, 2 * H)),
                _full((H, 2 * H)), _SMEM_SPEC],
      out_specs=[_rows(2 * H), _rows(H)],
      out_shape=[jax.ShapeDtypeStruct((N, 2 * H), F32),
                 jax.ShapeDtypeStruct((N, H), F32)],
  )(z2, g2.reshape(1, 2 * H), be2.reshape(1, 2 * H), Wl3,
    _finish_stats(st2, N * 2 * H))

  ap0, ap1 = _make_seg_sum(False)(p2.reshape(2 * N, 128), src, dst)

  z3, st3 = pl.pallas_call(
      _m3_body,
      grid=(GRID,),
      in_specs=[_rows(128), _rows(128), _rows(2 * H), _rows(1),
                _full((H, 2 * H)), _full((1, H))],
      out_specs=[_rows(H), _ST_SPEC],
      out_shape=[jax.ShapeDtypeStruct((N, H), F32), _ST_SHAPE],
      scratch_shapes=_SACC,
  )(ap0, ap1, h2, deg, Wr3, b3.reshape(1, H))

  out = pl.pallas_call(
      _f_body,
      grid=(GRID,),
      in_specs=[_rows(H), _full((1, H)), _full((1, H)), _SMEM_SPEC,
                _full((H // 2, H)), _full((1, H // 2)), _full((1, H // 2)),
                _full((1, H // 2)), _full((OUT, H // 2)), _full((1, OUT))],
      out_specs=_rows(OUT),
      out_shape=jax.ShapeDtypeStruct((N, OUT), F32),
  )(z3, g3.reshape(1, H), be3.reshape(1, H), _finish_stats(st3, N * H),
    W4, b4.reshape(1, H // 2), g4.reshape(1, H // 2), be4.reshape(1, H // 2),
    W5, b5.reshape(1, OUT))

  return out
